# baseline (speedup 1.0000x reference)
import functools

import jax
import jax.numpy as jnp
from jax import lax
from jax.experimental import pallas as pl
from jax.experimental.pallas import tpu as pltpu
from jax.experimental.pallas import tpu_sc as plsc

N_GROUP = 8
TOPK_GROUP = 4
TOP_K = 8
NUM_WORKERS = 32
N_CHUNK = 4
MASKED = -1.0e30
REMOVED = -2.0e30


def _logits_body(grid_n, tok_ref, wt_ref, ml_ref, accn_ref):
    i = pl.program_id(0)
    B = tok_ref.shape[0]
    E = wt_ref.shape[1]
    eg = E // N_GROUP
    NEGBIG = jnp.float32(-3.0e38)

    logits = jnp.dot(tok_ref[...], wt_ref[...], preferred_element_type=jnp.float32)
    lt = logits.T

    gm = jnp.concatenate(
        [jnp.max(lt[g * eg:(g + 1) * eg], axis=0, keepdims=True)
         for g in range(N_GROUP)], axis=0)
    grow = jax.lax.broadcasted_iota(jnp.int32, (N_GROUP, B), 0)
    gwork = gm
    gsel = jnp.zeros((N_GROUP, B), jnp.bool_)
    for _ in range(TOPK_GROUP):
        m = jnp.max(gwork, axis=0, keepdims=True)
        selg = jnp.min(jnp.where(gwork == m, grow, N_GROUP), axis=0, keepdims=True)
        hit = grow == selg
        gsel = gsel | hit
        gwork = jnp.where(hit, NEGBIG, gwork)

    ml_ref[0] = jnp.concatenate(
        [jnp.where(gsel[g:g + 1], lt[g * eg:(g + 1) * eg], jnp.float32(MASKED))
         for g in range(N_GROUP)], axis=0)

    scores = jax.nn.sigmoid(lt)
    ssum = jnp.maximum(jnp.sum(scores, axis=0, keepdims=True), jnp.float32(1e-9))

    @pl.when(i == 0)
    def _init():
        accn_ref[...] = jnp.zeros_like(accn_ref)

    accn_ref[...] += scores / ssum


def _route_body(tpw, ml_hbm, idx_hbm, w_hbm, accc_hbm, lv, idxv, wv, acccv):
    E = 64
    wid = lax.axis_index("s") * 2 + lax.axis_index("c")
    pltpu.sync_copy(ml_hbm.at[wid], lv)

    zero16 = jnp.zeros((16,), jnp.float32)
    for e in range(E):
        acccv[pl.ds(e * 16, 16)] = zero16
    lane = jax.lax.broadcasted_iota(jnp.int32, (16,), 0)
    ones16 = jnp.ones((16,), jnp.float32)
    rem16 = jnp.full((16,), REMOVED, jnp.float32)

    def chunk(j, carry):
        off = j * 16
        ms = []
        mis = []
        for _k in range(TOP_K):
            nodes = [(lv[pl.ds(e * tpw + off, 16)],
                      jnp.full((16,), e, jnp.int32)) for e in range(E)]
            while len(nodes) > 1:
                nxt = []
                for a, b in zip(nodes[0::2], nodes[1::2]):
                    t = b[0] > a[0]
                    nxt.append((jnp.where(t, b[0], a[0]),
                                jnp.where(t, b[1], a[1])))
                nodes = nxt
            m, mi = nodes[0]
            plsc.store_scatter(lv, [mi * tpw + (off + lane)], rem16)
            plsc.addupdate_scatter(acccv, [mi * 16 + lane], ones16)
            ms.append(m)
            mis.append(mi)
        ws = [jnp.float32(1.0) / (jnp.float32(1.0) + jnp.exp(-m)) for m in ms]
        denom = ws[0]
        for w in ws[1:]:
            denom = denom + w
        denom = jnp.maximum(denom, jnp.float32(1e-9))
        for k in range(TOP_K):
            idxv[pl.ds(k * tpw + off, 16)] = mis[k]
            wv[pl.ds(k * tpw + off, 16)] = ws[k] / denom
        return carry

    lax.fori_loop(0, tpw // 16, chunk, 0)

    pltpu.sync_copy(idxv, idx_hbm.at[wid])
    pltpu.sync_copy(wv, w_hbm.at[wid])
    pltpu.sync_copy(acccv, accc_hbm.at[wid])


def _aux_body(T, n_in, *refs):
    accn_refs = refs[:n_in]
    accc_refs = refs[n_in:2 * n_in]
    aux_ref = refs[2 * n_in]
    E = accn_refs[0].shape[0]
    a2 = jnp.zeros((E, 1), jnp.float32)
    for r in accn_refs:
        a2 = a2 + jnp.sum(r[...], axis=1, keepdims=True)
    s2 = jnp.zeros((E, 1), jnp.float32)
    for r in accc_refs:
        s1 = jnp.sum(r[...], axis=0)
        s2 = s2 + jnp.sum(s1, axis=1, keepdims=True)
    scale = jnp.float32(E) / (jnp.float32(T) * jnp.float32(T) * jnp.float32(TOP_K))
    aux_ref[...] = jnp.full((1, 1), jnp.sum(s2 * a2) * scale, jnp.float32)


def kernel(tokens, W):
    T, H = tokens.shape
    E = W.shape[0]
    B = 128
    tc = T // N_CHUNK
    grid_n = tc // B
    tpw = tc // NUM_WORKERS
    Wt = W.T

    mesh = plsc.VectorSubcoreMesh(core_axis_name="c", subcore_axis_name="s",
                                  num_cores=2, num_subcores=16)
    route = pl.kernel(
        functools.partial(_route_body, tpw),
        out_type=[
            jax.ShapeDtypeStruct((NUM_WORKERS, TOP_K * tpw), jnp.int32),
            jax.ShapeDtypeStruct((NUM_WORKERS, TOP_K * tpw), jnp.float32),
            jax.ShapeDtypeStruct((NUM_WORKERS, E * 16), jnp.float32),
        ],
        mesh=mesh,
        scratch_types=[
            pltpu.VMEM((E * tpw,), jnp.float32),
            pltpu.VMEM((TOP_K * tpw,), jnp.int32),
            pltpu.VMEM((TOP_K * tpw,), jnp.float32),
            pltpu.VMEM((E * 16,), jnp.float32),
        ],
        compiler_params=pltpu.CompilerParams(needs_layout_passes=False),
    )

    accns = []
    acccs = []
    idxs = []
    wss = []
    for ch in range(N_CHUNK):
        ml, accn = pl.pallas_call(
            functools.partial(_logits_body, grid_n),
            grid=(grid_n,),
            in_specs=[
                pl.BlockSpec((B, H), lambda i, ch=ch: (ch * grid_n + i, 0)),
                pl.BlockSpec((H, E), lambda i: (0, 0)),
            ],
            out_specs=[
                pl.BlockSpec((1, E, B), lambda i: (i, 0, 0)),
                pl.BlockSpec((E, B), lambda i: (0, 0)),
            ],
            out_shape=[
                jax.ShapeDtypeStruct((NUM_WORKERS, E, tpw), jnp.float32),
                jax.ShapeDtypeStruct((E, B), jnp.float32),
            ],
            compiler_params=pltpu.CompilerParams(
                dimension_semantics=("arbitrary",),
            ),
        )(tokens, Wt)
        idx3, w3, accc = route(ml.reshape(NUM_WORKERS, E * tpw))
        accns.append(accn)
        acccs.append(accc.reshape(NUM_WORKERS, E, 16))
        idxs.append(jnp.transpose(idx3.reshape(NUM_WORKERS, TOP_K, tpw),
                                  (0, 2, 1)).reshape(tc, TOP_K))
        wss.append(jnp.transpose(w3.reshape(NUM_WORKERS, TOP_K, tpw),
                                 (0, 2, 1)).reshape(tc, TOP_K))

    aux = pl.pallas_call(
        functools.partial(_aux_body, T, N_CHUNK),
        out_shape=jax.ShapeDtypeStruct((1, 1), jnp.float32),
    )(*accns, *acccs)

    idx = jnp.concatenate(idxs, axis=0)
    w = jnp.concatenate(wss, axis=0)
    return (idx, w, aux[0, 0])


# speedup vs baseline: 1.0484x; 1.0484x over previous
import functools

import jax
import jax.numpy as jnp
from jax import lax
from jax.experimental import pallas as pl
from jax.experimental.pallas import tpu as pltpu
from jax.experimental.pallas import tpu_sc as plsc

N_GROUP = 8
TOPK_GROUP = 4
TOP_K = 8
NUM_WORKERS = 32
N_CHUNK = 1
MASKED = -1.0e30
REMOVED = -2.0e30


def _logits_body(grid_n, tok_ref, wt_ref, ml_ref, accn_ref):
    i = pl.program_id(0)
    B = tok_ref.shape[0]
    E = wt_ref.shape[1]
    eg = E // N_GROUP
    NEGBIG = jnp.float32(-3.0e38)

    logits = jnp.dot(tok_ref[...], wt_ref[...], preferred_element_type=jnp.float32)
    lt = logits.T

    gm = jnp.concatenate(
        [jnp.max(lt[g * eg:(g + 1) * eg], axis=0, keepdims=True)
         for g in range(N_GROUP)], axis=0)
    grow = jax.lax.broadcasted_iota(jnp.int32, (N_GROUP, B), 0)
    gwork = gm
    gsel = jnp.zeros((N_GROUP, B), jnp.bool_)
    for _ in range(TOPK_GROUP):
        m = jnp.max(gwork, axis=0, keepdims=True)
        selg = jnp.min(jnp.where(gwork == m, grow, N_GROUP), axis=0, keepdims=True)
        hit = grow == selg
        gsel = gsel | hit
        gwork = jnp.where(hit, NEGBIG, gwork)

    ml_ref[0] = jnp.concatenate(
        [jnp.where(gsel[g:g + 1], lt[g * eg:(g + 1) * eg], jnp.float32(MASKED))
         for g in range(N_GROUP)], axis=0)

    scores = jax.nn.sigmoid(lt)
    ssum = jnp.maximum(jnp.sum(scores, axis=0, keepdims=True), jnp.float32(1e-9))

    @pl.when(i == 0)
    def _init():
        accn_ref[...] = jnp.zeros_like(accn_ref)

    accn_ref[...] += scores / ssum


def _route_body(tpw, ml_hbm, idx_hbm, w_hbm, accc_hbm, lv, idxv, wv, acccv):
    E = 64
    wid = lax.axis_index("s") * 2 + lax.axis_index("c")
    pltpu.sync_copy(ml_hbm.at[wid], lv)

    zero16 = jnp.zeros((16,), jnp.float32)
    for e in range(E):
        acccv[pl.ds(e * 16, 16)] = zero16
    lane = jax.lax.broadcasted_iota(jnp.int32, (16,), 0)
    ones16 = jnp.ones((16,), jnp.float32)
    rem16 = jnp.full((16,), REMOVED, jnp.float32)

    def chunk(j, carry):
        off = j * 16
        ms = []
        mis = []
        for _k in range(TOP_K):
            nodes = [(lv[pl.ds(e * tpw + off, 16)],
                      jnp.full((16,), e, jnp.int32)) for e in range(E)]
            while len(nodes) > 1:
                nxt = []
                for a, b in zip(nodes[0::2], nodes[1::2]):
                    t = b[0] > a[0]
                    nxt.append((jnp.where(t, b[0], a[0]),
                                jnp.where(t, b[1], a[1])))
                nodes = nxt
            m, mi = nodes[0]
            plsc.store_scatter(lv, [mi * tpw + (off + lane)], rem16)
            plsc.addupdate_scatter(acccv, [mi * 16 + lane], ones16)
            ms.append(m)
            mis.append(mi)
        ws = [jnp.float32(1.0) / (jnp.float32(1.0) + jnp.exp(-m)) for m in ms]
        denom = ws[0]
        for w in ws[1:]:
            denom = denom + w
        denom = jnp.maximum(denom, jnp.float32(1e-9))
        for k in range(TOP_K):
            idxv[pl.ds(k * tpw + off, 16)] = mis[k]
            wv[pl.ds(k * tpw + off, 16)] = ws[k] / denom
        return carry

    lax.fori_loop(0, tpw // 16, chunk, 0)

    pltpu.sync_copy(idxv, idx_hbm.at[wid])
    pltpu.sync_copy(wv, w_hbm.at[wid])
    pltpu.sync_copy(acccv, accc_hbm.at[wid])


def _aux_body(T, n_in, *refs):
    accn_refs = refs[:n_in]
    accc_refs = refs[n_in:2 * n_in]
    aux_ref = refs[2 * n_in]
    E = accn_refs[0].shape[0]
    a2 = jnp.zeros((E, 1), jnp.float32)
    for r in accn_refs:
        a2 = a2 + jnp.sum(r[...], axis=1, keepdims=True)
    s2 = jnp.zeros((E, 1), jnp.float32)
    for r in accc_refs:
        s1 = jnp.sum(r[...], axis=0)
        s2 = s2 + jnp.sum(s1, axis=1, keepdims=True)
    scale = jnp.float32(E) / (jnp.float32(T) * jnp.float32(T) * jnp.float32(TOP_K))
    aux_ref[...] = jnp.full((1, 1), jnp.sum(s2 * a2) * scale, jnp.float32)


def kernel(tokens, W):
    T, H = tokens.shape
    E = W.shape[0]
    B = 128
    tc = T // N_CHUNK
    grid_n = tc // B
    tpw = tc // NUM_WORKERS
    Wt = W.T

    mesh = plsc.VectorSubcoreMesh(core_axis_name="c", subcore_axis_name="s",
                                  num_cores=2, num_subcores=16)
    route = pl.kernel(
        functools.partial(_route_body, tpw),
        out_type=[
            jax.ShapeDtypeStruct((NUM_WORKERS, TOP_K * tpw), jnp.int32),
            jax.ShapeDtypeStruct((NUM_WORKERS, TOP_K * tpw), jnp.float32),
            jax.ShapeDtypeStruct((NUM_WORKERS, E * 16), jnp.float32),
        ],
        mesh=mesh,
        scratch_types=[
            pltpu.VMEM((E * tpw,), jnp.float32),
            pltpu.VMEM((TOP_K * tpw,), jnp.int32),
            pltpu.VMEM((TOP_K * tpw,), jnp.float32),
            pltpu.VMEM((E * 16,), jnp.float32),
        ],
        compiler_params=pltpu.CompilerParams(needs_layout_passes=False),
    )

    accns = []
    acccs = []
    idxs = []
    wss = []
    for ch in range(N_CHUNK):
        ml, accn = pl.pallas_call(
            functools.partial(_logits_body, grid_n),
            grid=(grid_n,),
            in_specs=[
                pl.BlockSpec((B, H), lambda i, ch=ch: (ch * grid_n + i, 0)),
                pl.BlockSpec((H, E), lambda i: (0, 0)),
            ],
            out_specs=[
                pl.BlockSpec((1, E, B), lambda i: (i, 0, 0)),
                pl.BlockSpec((E, B), lambda i: (0, 0)),
            ],
            out_shape=[
                jax.ShapeDtypeStruct((NUM_WORKERS, E, tpw), jnp.float32),
                jax.ShapeDtypeStruct((E, B), jnp.float32),
            ],
            compiler_params=pltpu.CompilerParams(
                dimension_semantics=("arbitrary",),
            ),
        )(tokens, Wt)
        idx3, w3, accc = route(ml.reshape(NUM_WORKERS, E * tpw))
        accns.append(accn)
        acccs.append(accc.reshape(NUM_WORKERS, E, 16))
        idxs.append(jnp.transpose(idx3.reshape(NUM_WORKERS, TOP_K, tpw),
                                  (0, 2, 1)).reshape(tc, TOP_K))
        wss.append(jnp.transpose(w3.reshape(NUM_WORKERS, TOP_K, tpw),
                                 (0, 2, 1)).reshape(tc, TOP_K))

    aux = pl.pallas_call(
        functools.partial(_aux_body, T, N_CHUNK),
        out_shape=jax.ShapeDtypeStruct((1, 1), jnp.float32),
    )(*accns, *acccs)

    idx = jnp.concatenate(idxs, axis=0)
    w = jnp.concatenate(wss, axis=0)
    return (idx, w, aux[0, 0])


# speedup vs baseline: 1.3760x; 1.3125x over previous
import functools

import jax
import jax.numpy as jnp
from jax import lax
from jax.experimental import pallas as pl
from jax.experimental.pallas import tpu as pltpu
from jax.experimental.pallas import tpu_sc as plsc

N_GROUP = 8
TOPK_GROUP = 4
TOP_K = 8
NUM_WORKERS = 32
N_CHUNK = 4
MASKED = -1.0e30
REMOVED = -2.0e30


def _logits_body(grid_n, tok_ref, wt_ref, ml_ref, accn_ref):
    i = pl.program_id(0)
    B = tok_ref.shape[0]
    E = wt_ref.shape[1]
    eg = E // N_GROUP
    NEGBIG = jnp.float32(-3.0e38)

    logits = jnp.dot(tok_ref[...], wt_ref[...], preferred_element_type=jnp.float32)
    lt = logits.T

    gm = jnp.concatenate(
        [jnp.max(lt[g * eg:(g + 1) * eg], axis=0, keepdims=True)
         for g in range(N_GROUP)], axis=0)
    grow = jax.lax.broadcasted_iota(jnp.int32, (N_GROUP, B), 0)
    gwork = gm
    gsel = jnp.zeros((N_GROUP, B), jnp.bool_)
    for _ in range(TOPK_GROUP):
        m = jnp.max(gwork, axis=0, keepdims=True)
        selg = jnp.min(jnp.where(gwork == m, grow, N_GROUP), axis=0, keepdims=True)
        hit = grow == selg
        gsel = gsel | hit
        gwork = jnp.where(hit, NEGBIG, gwork)

    masked = jnp.concatenate(
        [jnp.where(gsel[g:g + 1], lt[g * eg:(g + 1) * eg], jnp.float32(MASKED))
         for g in range(N_GROUP)], axis=0)
    spb = ml_ref.shape[0]
    tpw = B // spb
    for wslab in range(spb):
        ml_ref[wslab] = masked[:, wslab * tpw:(wslab + 1) * tpw]

    scores = jax.nn.sigmoid(lt)
    ssum = jnp.maximum(jnp.sum(scores, axis=0, keepdims=True), jnp.float32(1e-9))

    @pl.when(i == 0)
    def _init():
        accn_ref[...] = jnp.zeros_like(accn_ref)

    accn_ref[...] += scores / ssum


def _route_body(tpw, ml_hbm, idx_hbm, w_hbm, accc_hbm, lv, idxv, wv, acccv):
    E = 64
    wid = lax.axis_index("s") * 2 + lax.axis_index("c")
    pltpu.sync_copy(ml_hbm.at[wid], lv)

    zero16 = jnp.zeros((16,), jnp.float32)
    for e in range(E):
        acccv[pl.ds(e * 16, 16)] = zero16
    lane = jax.lax.broadcasted_iota(jnp.int32, (16,), 0)
    ones16 = jnp.ones((16,), jnp.float32)
    rem16 = jnp.full((16,), REMOVED, jnp.float32)

    def chunk(j, carry):
        off = j * 16
        ms = []
        mis = []
        for _k in range(TOP_K):
            nodes = [(lv[pl.ds(e * tpw + off, 16)],
                      jnp.full((16,), e, jnp.int32)) for e in range(E)]
            while len(nodes) > 1:
                nxt = []
                for a, b in zip(nodes[0::2], nodes[1::2]):
                    t = b[0] > a[0]
                    nxt.append((jnp.where(t, b[0], a[0]),
                                jnp.where(t, b[1], a[1])))
                nodes = nxt
            m, mi = nodes[0]
            plsc.store_scatter(lv, [mi * tpw + (off + lane)], rem16)
            plsc.addupdate_scatter(acccv, [mi * 16 + lane], ones16)
            ms.append(m)
            mis.append(mi)
        ws = [jnp.float32(1.0) / (jnp.float32(1.0) + jnp.exp(-m)) for m in ms]
        denom = ws[0]
        for w in ws[1:]:
            denom = denom + w
        denom = jnp.maximum(denom, jnp.float32(1e-9))
        for k in range(TOP_K):
            idxv[pl.ds(k * tpw + off, 16)] = mis[k]
            wv[pl.ds(k * tpw + off, 16)] = ws[k] / denom
        return carry

    lax.fori_loop(0, tpw // 16, chunk, 0)

    pltpu.sync_copy(idxv, idx_hbm.at[wid])
    pltpu.sync_copy(wv, w_hbm.at[wid])
    pltpu.sync_copy(acccv, accc_hbm.at[wid])


def _aux_body(T, n_in, *refs):
    accn_refs = refs[:n_in]
    accc_refs = refs[n_in:2 * n_in]
    aux_ref = refs[2 * n_in]
    E = accn_refs[0].shape[0]
    a2 = jnp.zeros((E, 1), jnp.float32)
    for r in accn_refs:
        a2 = a2 + jnp.sum(r[...], axis=1, keepdims=True)
    s2 = jnp.zeros((E, 1), jnp.float32)
    for r in accc_refs:
        s1 = jnp.sum(r[...], axis=0)
        s2 = s2 + jnp.sum(s1, axis=1, keepdims=True)
    scale = jnp.float32(E) / (jnp.float32(T) * jnp.float32(T) * jnp.float32(TOP_K))
    aux_ref[...] = jnp.full((1, 1), jnp.sum(s2 * a2) * scale, jnp.float32)


def kernel(tokens, W):
    T, H = tokens.shape
    E = W.shape[0]
    B = 512
    tc = T // N_CHUNK
    grid_n = tc // B
    tpw = tc // NUM_WORKERS
    spb = B // tpw
    Wt = W.T

    mesh = plsc.VectorSubcoreMesh(core_axis_name="c", subcore_axis_name="s",
                                  num_cores=2, num_subcores=16)
    route = pl.kernel(
        functools.partial(_route_body, tpw),
        out_type=[
            jax.ShapeDtypeStruct((NUM_WORKERS, TOP_K * tpw), jnp.int32),
            jax.ShapeDtypeStruct((NUM_WORKERS, TOP_K * tpw), jnp.float32),
            jax.ShapeDtypeStruct((NUM_WORKERS, E * 16), jnp.float32),
        ],
        mesh=mesh,
        scratch_types=[
            pltpu.VMEM((E * tpw,), jnp.float32),
            pltpu.VMEM((TOP_K * tpw,), jnp.int32),
            pltpu.VMEM((TOP_K * tpw,), jnp.float32),
            pltpu.VMEM((E * 16,), jnp.float32),
        ],
        compiler_params=pltpu.CompilerParams(needs_layout_passes=False),
    )

    accns = []
    acccs = []
    idxs = []
    wss = []
    for ch in range(N_CHUNK):
        ml, accn = pl.pallas_call(
            functools.partial(_logits_body, grid_n),
            grid=(grid_n,),
            in_specs=[
                pl.BlockSpec((B, H), lambda i, ch=ch: (ch * grid_n + i, 0)),
                pl.BlockSpec((H, E), lambda i: (0, 0)),
            ],
            out_specs=[
                pl.BlockSpec((spb, E, tpw), lambda i: (i, 0, 0)),
                pl.BlockSpec((E, B), lambda i: (0, 0)),
            ],
            out_shape=[
                jax.ShapeDtypeStruct((NUM_WORKERS, E, tpw), jnp.float32),
                jax.ShapeDtypeStruct((E, B), jnp.float32),
            ],
            compiler_params=pltpu.CompilerParams(
                dimension_semantics=("arbitrary",),
            ),
        )(tokens, Wt)
        idx3, w3, accc = route(ml.reshape(NUM_WORKERS, E * tpw))
        accns.append(accn)
        acccs.append(accc.reshape(NUM_WORKERS, E, 16))
        idxs.append(jnp.transpose(idx3.reshape(NUM_WORKERS, TOP_K, tpw),
                                  (0, 2, 1)).reshape(tc, TOP_K))
        wss.append(jnp.transpose(w3.reshape(NUM_WORKERS, TOP_K, tpw),
                                 (0, 2, 1)).reshape(tc, TOP_K))

    aux = pl.pallas_call(
        functools.partial(_aux_body, T, N_CHUNK),
        out_shape=jax.ShapeDtypeStruct((1, 1), jnp.float32),
    )(*accns, *acccs)

    idx = jnp.concatenate(idxs, axis=0)
    w = jnp.concatenate(wss, axis=0)
    return (idx, w, aux[0, 0])


# speedup vs baseline: 1.6302x; 1.1847x over previous
import functools

import jax
import jax.numpy as jnp
from jax import lax
from jax.experimental import pallas as pl
from jax.experimental.pallas import tpu as pltpu
from jax.experimental.pallas import tpu_sc as plsc

N_GROUP = 8
TOPK_GROUP = 4
TOP_K = 8
NUM_WORKERS = 32
N_CHUNK = 1
MASKED = -1.0e30
REMOVED = -2.0e30


def _logits_body(grid_n, tok_ref, wt_ref, ml_ref, accn_ref):
    i = pl.program_id(0)
    B = tok_ref.shape[0]
    E = wt_ref.shape[1]
    eg = E // N_GROUP
    NEGBIG = jnp.float32(-3.0e38)

    logits = jnp.dot(tok_ref[...], wt_ref[...], preferred_element_type=jnp.float32)
    lt = logits.T

    gm = jnp.concatenate(
        [jnp.max(lt[g * eg:(g + 1) * eg], axis=0, keepdims=True)
         for g in range(N_GROUP)], axis=0)
    grow = jax.lax.broadcasted_iota(jnp.int32, (N_GROUP, B), 0)
    gwork = gm
    gsel = jnp.zeros((N_GROUP, B), jnp.bool_)
    for _ in range(TOPK_GROUP):
        m = jnp.max(gwork, axis=0, keepdims=True)
        selg = jnp.min(jnp.where(gwork == m, grow, N_GROUP), axis=0, keepdims=True)
        hit = grow == selg
        gsel = gsel | hit
        gwork = jnp.where(hit, NEGBIG, gwork)

    masked = jnp.concatenate(
        [jnp.where(gsel[g:g + 1], lt[g * eg:(g + 1) * eg], jnp.float32(MASKED))
         for g in range(N_GROUP)], axis=0)
    spb = ml_ref.shape[0]
    tpw = B // spb
    for wslab in range(spb):
        ml_ref[wslab] = masked[:, wslab * tpw:(wslab + 1) * tpw]

    scores = jax.nn.sigmoid(lt)
    ssum = jnp.maximum(jnp.sum(scores, axis=0, keepdims=True), jnp.float32(1e-9))

    @pl.when(i == 0)
    def _init():
        accn_ref[...] = jnp.zeros_like(accn_ref)

    accn_ref[...] += scores / ssum


def _route_body(tpw, ml_hbm, idx_hbm, w_hbm, accc_hbm, lv, idxv, wv, acccv):
    E = 64
    wid = lax.axis_index("s") * 2 + lax.axis_index("c")
    pltpu.sync_copy(ml_hbm.at[wid], lv)

    zero16 = jnp.zeros((16,), jnp.float32)
    for e in range(E):
        acccv[pl.ds(e * 16, 16)] = zero16
    lane = jax.lax.broadcasted_iota(jnp.int32, (16,), 0)
    ones16 = jnp.ones((16,), jnp.float32)
    rem16 = jnp.full((16,), REMOVED, jnp.float32)

    def tourney(nodes):
        while len(nodes) > 1:
            nxt = []
            for a, b in zip(nodes[0::2], nodes[1::2]):
                t = b[0] > a[0]
                nxt.append((jnp.where(t, b[0], a[0]),
                            jnp.where(t, b[1], a[1])))
            nodes = nxt
        return nodes[0]

    def chunk(j, carry):
        off = j * 16
        ms = []
        mis = []
        gvals = []
        gidxs = []
        for g in range(N_GROUP):
            gv, gi = tourney(
                [(lv[pl.ds((g * 8 + r) * tpw + off, 16)],
                  jnp.full((16,), g * 8 + r, jnp.int32)) for r in range(8)])
            gvals.append(gv)
            gidxs.append(gi)
        for _k in range(TOP_K):
            m, mi = tourney(list(zip(gvals, gidxs)))
            plsc.store_scatter(lv, [mi * tpw + (off + lane)], rem16)
            plsc.addupdate_scatter(acccv, [mi * 16 + lane], ones16)
            ms.append(m)
            mis.append(mi)
            gwin = lax.shift_right_logical(mi, 3)
            gw8 = gwin * 8
            base = gwin * (8 * tpw) + (off + lane)
            nv, ni = tourney(
                [(plsc.load_gather(lv, [base + r * tpw]), gw8 + r)
                 for r in range(8)])
            for g in range(N_GROUP):
                isg = gwin == g
                gvals[g] = jnp.where(isg, nv, gvals[g])
                gidxs[g] = jnp.where(isg, ni, gidxs[g])
        ws = [jnp.float32(1.0) / (jnp.float32(1.0) + jnp.exp(-m)) for m in ms]
        denom = ws[0]
        for w in ws[1:]:
            denom = denom + w
        denom = jnp.maximum(denom, jnp.float32(1e-9))
        for k in range(TOP_K):
            idxv[pl.ds(k * tpw + off, 16)] = mis[k]
            wv[pl.ds(k * tpw + off, 16)] = ws[k] / denom
        return carry

    lax.fori_loop(0, tpw // 16, chunk, 0)

    pltpu.sync_copy(idxv, idx_hbm.at[wid])
    pltpu.sync_copy(wv, w_hbm.at[wid])
    pltpu.sync_copy(acccv, accc_hbm.at[wid])


def _aux_body(T, n_in, *refs):
    accn_refs = refs[:n_in]
    accc_refs = refs[n_in:2 * n_in]
    aux_ref = refs[2 * n_in]
    E = accn_refs[0].shape[0]
    a2 = jnp.zeros((E, 1), jnp.float32)
    for r in accn_refs:
        a2 = a2 + jnp.sum(r[...], axis=1, keepdims=True)
    s2 = jnp.zeros((E, 1), jnp.float32)
    for r in accc_refs:
        s1 = jnp.sum(r[...], axis=0)
        s2 = s2 + jnp.sum(s1, axis=1, keepdims=True)
    scale = jnp.float32(E) / (jnp.float32(T) * jnp.float32(T) * jnp.float32(TOP_K))
    aux_ref[...] = jnp.full((1, 1), jnp.sum(s2 * a2) * scale, jnp.float32)


def kernel(tokens, W):
    T, H = tokens.shape
    E = W.shape[0]
    B = 512
    tc = T // N_CHUNK
    grid_n = tc // B
    tpw = tc // NUM_WORKERS
    spb = B // tpw
    Wt = W.T

    mesh = plsc.VectorSubcoreMesh(core_axis_name="c", subcore_axis_name="s",
                                  num_cores=2, num_subcores=16)
    route = pl.kernel(
        functools.partial(_route_body, tpw),
        out_type=[
            jax.ShapeDtypeStruct((NUM_WORKERS, TOP_K * tpw), jnp.int32),
            jax.ShapeDtypeStruct((NUM_WORKERS, TOP_K * tpw), jnp.float32),
            jax.ShapeDtypeStruct((NUM_WORKERS, E * 16), jnp.float32),
        ],
        mesh=mesh,
        scratch_types=[
            pltpu.VMEM((E * tpw,), jnp.float32),
            pltpu.VMEM((TOP_K * tpw,), jnp.int32),
            pltpu.VMEM((TOP_K * tpw,), jnp.float32),
            pltpu.VMEM((E * 16,), jnp.float32),
        ],
        compiler_params=pltpu.CompilerParams(needs_layout_passes=False),
    )

    accns = []
    acccs = []
    idxs = []
    wss = []
    for ch in range(N_CHUNK):
        ml, accn = pl.pallas_call(
            functools.partial(_logits_body, grid_n),
            grid=(grid_n,),
            in_specs=[
                pl.BlockSpec((B, H), lambda i, ch=ch: (ch * grid_n + i, 0)),
                pl.BlockSpec((H, E), lambda i: (0, 0)),
            ],
            out_specs=[
                pl.BlockSpec((spb, E, tpw), lambda i: (i, 0, 0)),
                pl.BlockSpec((E, B), lambda i: (0, 0)),
            ],
            out_shape=[
                jax.ShapeDtypeStruct((NUM_WORKERS, E, tpw), jnp.float32),
                jax.ShapeDtypeStruct((E, B), jnp.float32),
            ],
            compiler_params=pltpu.CompilerParams(
                dimension_semantics=("arbitrary",),
            ),
        )(tokens, Wt)
        idx3, w3, accc = route(ml.reshape(NUM_WORKERS, E * tpw))
        accns.append(accn)
        acccs.append(accc.reshape(NUM_WORKERS, E, 16))
        idxs.append(jnp.transpose(idx3.reshape(NUM_WORKERS, TOP_K, tpw),
                                  (0, 2, 1)).reshape(tc, TOP_K))
        wss.append(jnp.transpose(w3.reshape(NUM_WORKERS, TOP_K, tpw),
                                 (0, 2, 1)).reshape(tc, TOP_K))

    aux = pl.pallas_call(
        functools.partial(_aux_body, T, N_CHUNK),
        out_shape=jax.ShapeDtypeStruct((1, 1), jnp.float32),
    )(*accns, *acccs)

    idx = jnp.concatenate(idxs, axis=0)
    w = jnp.concatenate(wss, axis=0)
    return (idx, w, aux[0, 0])


# speedup vs baseline: 1.6320x; 1.0011x over previous
import functools

import jax
import jax.numpy as jnp
from jax import lax
from jax.experimental import pallas as pl
from jax.experimental.pallas import tpu as pltpu
from jax.experimental.pallas import tpu_sc as plsc

N_GROUP = 8
TOPK_GROUP = 4
TOP_K = 8
NUM_WORKERS = 32
N_CHUNK = 1
MASKED = -1.0e30
REMOVED = -2.0e30


def _logits_body(grid_n, tok_ref, wt_ref, ml_ref, accn_ref):
    i = pl.program_id(0)
    B = tok_ref.shape[0]
    E = wt_ref.shape[1]
    eg = E // N_GROUP
    NEGBIG = jnp.float32(-3.0e38)

    logits = jnp.dot(tok_ref[...], wt_ref[...], preferred_element_type=jnp.float32)
    lt = logits.T

    gm = jnp.concatenate(
        [jnp.max(lt[g * eg:(g + 1) * eg], axis=0, keepdims=True)
         for g in range(N_GROUP)], axis=0)
    grow = jax.lax.broadcasted_iota(jnp.int32, (N_GROUP, B), 0)
    gwork = gm
    gsel = jnp.zeros((N_GROUP, B), jnp.bool_)
    for _ in range(TOPK_GROUP):
        m = jnp.max(gwork, axis=0, keepdims=True)
        selg = jnp.min(jnp.where(gwork == m, grow, N_GROUP), axis=0, keepdims=True)
        hit = grow == selg
        gsel = gsel | hit
        gwork = jnp.where(hit, NEGBIG, gwork)

    masked = jnp.concatenate(
        [jnp.where(gsel[g:g + 1], lt[g * eg:(g + 1) * eg], jnp.float32(MASKED))
         for g in range(N_GROUP)], axis=0)
    spb = ml_ref.shape[0]
    tpw = B // spb
    for wslab in range(spb):
        ml_ref[wslab] = masked[:, wslab * tpw:(wslab + 1) * tpw]

    scores = jax.nn.sigmoid(lt)
    ssum = jnp.maximum(jnp.sum(scores, axis=0, keepdims=True), jnp.float32(1e-9))

    @pl.when(i == 0)
    def _init():
        accn_ref[...] = jnp.zeros_like(accn_ref)

    accn_ref[...] += scores / ssum


def _route_body(tpw, ml_hbm, idx_hbm, w_hbm, accc_hbm, lv, idxv, wv, acccv):
    E = 64
    wid = lax.axis_index("s") * 2 + lax.axis_index("c")
    pltpu.sync_copy(ml_hbm.at[wid], lv)

    zero16 = jnp.zeros((16,), jnp.float32)
    for e in range(E):
        acccv[pl.ds(e * 16, 16)] = zero16
    lane = jax.lax.broadcasted_iota(jnp.int32, (16,), 0)
    ones16 = jnp.ones((16,), jnp.float32)
    rem16 = jnp.full((16,), REMOVED, jnp.float32)

    def tourney(nodes):
        while len(nodes) > 1:
            nxt = []
            for a, b in zip(nodes[0::2], nodes[1::2]):
                t = b[0] > a[0]
                nxt.append((jnp.where(t, b[0], a[0]),
                            jnp.where(t, b[1], a[1])))
            nodes = nxt
        return nodes[0]

    def chunk(j, carry):
        off = j * 16
        ms = []
        mis = []
        gvals = []
        gidxs = []
        for g in range(N_GROUP):
            gv, gi = tourney(
                [(lv[pl.ds((g * 8 + r) * tpw + off, 16)],
                  jnp.full((16,), g * 8 + r, jnp.int32)) for r in range(8)])
            gvals.append(gv)
            gidxs.append(gi)
        for _k in range(TOP_K):
            m, mi = tourney(list(zip(gvals, gidxs)))
            plsc.store_scatter(lv, [mi * tpw + (off + lane)], rem16)
            plsc.addupdate_scatter(acccv, [mi * 16 + lane], ones16)
            ms.append(m)
            mis.append(mi)
            if _k == TOP_K - 1:
                break
            gwin = lax.shift_right_logical(mi, 3)
            gw8 = gwin * 8
            base = gwin * (8 * tpw) + (off + lane)
            nv, ni = tourney(
                [(plsc.load_gather(lv, [base + r * tpw]), gw8 + r)
                 for r in range(8)])
            for g in range(N_GROUP):
                isg = gwin == g
                gvals[g] = jnp.where(isg, nv, gvals[g])
                gidxs[g] = jnp.where(isg, ni, gidxs[g])
        ws = [jnp.float32(1.0) / (jnp.float32(1.0) + jnp.exp(-m)) for m in ms]
        denom = ws[0]
        for w in ws[1:]:
            denom = denom + w
        inv = jnp.float32(1.0) / jnp.maximum(denom, jnp.float32(1e-9))
        for k in range(TOP_K):
            idxv[pl.ds(k * tpw + off, 16)] = mis[k]
            wv[pl.ds(k * tpw + off, 16)] = ws[k] * inv
        return carry

    lax.fori_loop(0, tpw // 16, chunk, 0)

    pltpu.sync_copy(idxv, idx_hbm.at[wid])
    pltpu.sync_copy(wv, w_hbm.at[wid])
    pltpu.sync_copy(acccv, accc_hbm.at[wid])


def _aux_body(T, n_in, *refs):
    accn_refs = refs[:n_in]
    accc_refs = refs[n_in:2 * n_in]
    aux_ref = refs[2 * n_in]
    E = accn_refs[0].shape[0]
    a2 = jnp.zeros((E, 1), jnp.float32)
    for r in accn_refs:
        a2 = a2 + jnp.sum(r[...], axis=1, keepdims=True)
    s2 = jnp.zeros((E, 1), jnp.float32)
    for r in accc_refs:
        s1 = jnp.sum(r[...], axis=0)
        s2 = s2 + jnp.sum(s1, axis=1, keepdims=True)
    scale = jnp.float32(E) / (jnp.float32(T) * jnp.float32(T) * jnp.float32(TOP_K))
    aux_ref[...] = jnp.full((1, 1), jnp.sum(s2 * a2) * scale, jnp.float32)


def kernel(tokens, W):
    T, H = tokens.shape
    E = W.shape[0]
    B = 512
    tc = T // N_CHUNK
    grid_n = tc // B
    tpw = tc // NUM_WORKERS
    spb = B // tpw
    Wt = W.T

    mesh = plsc.VectorSubcoreMesh(core_axis_name="c", subcore_axis_name="s",
                                  num_cores=2, num_subcores=16)
    route = pl.kernel(
        functools.partial(_route_body, tpw),
        out_type=[
            jax.ShapeDtypeStruct((NUM_WORKERS, TOP_K * tpw), jnp.int32),
            jax.ShapeDtypeStruct((NUM_WORKERS, TOP_K * tpw), jnp.float32),
            jax.ShapeDtypeStruct((NUM_WORKERS, E * 16), jnp.float32),
        ],
        mesh=mesh,
        scratch_types=[
            pltpu.VMEM((E * tpw,), jnp.float32),
            pltpu.VMEM((TOP_K * tpw,), jnp.int32),
            pltpu.VMEM((TOP_K * tpw,), jnp.float32),
            pltpu.VMEM((E * 16,), jnp.float32),
        ],
        compiler_params=pltpu.CompilerParams(needs_layout_passes=False),
    )

    accns = []
    acccs = []
    idxs = []
    wss = []
    for ch in range(N_CHUNK):
        ml, accn = pl.pallas_call(
            functools.partial(_logits_body, grid_n),
            grid=(grid_n,),
            in_specs=[
                pl.BlockSpec((B, H), lambda i, ch=ch: (ch * grid_n + i, 0)),
                pl.BlockSpec((H, E), lambda i: (0, 0)),
            ],
            out_specs=[
                pl.BlockSpec((spb, E, tpw), lambda i: (i, 0, 0)),
                pl.BlockSpec((E, B), lambda i: (0, 0)),
            ],
            out_shape=[
                jax.ShapeDtypeStruct((NUM_WORKERS, E, tpw), jnp.float32),
                jax.ShapeDtypeStruct((E, B), jnp.float32),
            ],
            compiler_params=pltpu.CompilerParams(
                dimension_semantics=("arbitrary",),
            ),
        )(tokens, Wt)
        idx3, w3, accc = route(ml.reshape(NUM_WORKERS, E * tpw))
        accns.append(accn)
        acccs.append(accc.reshape(NUM_WORKERS, E, 16))
        idxs.append(jnp.transpose(idx3.reshape(NUM_WORKERS, TOP_K, tpw),
                                  (0, 2, 1)).reshape(tc, TOP_K))
        wss.append(jnp.transpose(w3.reshape(NUM_WORKERS, TOP_K, tpw),
                                 (0, 2, 1)).reshape(tc, TOP_K))

    aux = pl.pallas_call(
        functools.partial(_aux_body, T, N_CHUNK),
        out_shape=jax.ShapeDtypeStruct((1, 1), jnp.float32),
    )(*accns, *acccs)

    idx = jnp.concatenate(idxs, axis=0)
    w = jnp.concatenate(wss, axis=0)
    return (idx, w, aux[0, 0])
